# Initial kernel scaffold; baseline (speedup 1.0000x reference)
import jax
import jax.numpy as jnp
from jax.experimental import pallas as pl


def kernel(x, w_exp, b_exp, w_dw, b_dw, w_se1, b_se1, w_se2, b_se2, w_proj, b_proj):
    raise NotImplementedError("write your pallas kernel here")


# speedup vs baseline: 1.3111x; 1.3111x over previous
import functools

import jax
import jax.numpy as jnp
from jax import lax
from jax.experimental import pallas as pl
from jax.experimental.pallas import tpu as pltpu


def _silu(v):
    return v * jax.nn.sigmoid(v)


def _fused_kernel(x_ref, we_ref, be_ref, wd_ref, bd_ref,
                  ws1_ref, bs1_ref, ws2_ref, bs2_ref,
                  wp_ref, bp_ref, o_ref, *, G, H, W, K):
    HW = H * W
    Cexp = we_ref.shape[1]
    pad = (K - 1) // 2
    we = we_ref[...]
    be = be_ref[...]
    wd = wd_ref[...]
    bd = bd_ref[...]

    acts = []
    pooled = []
    for g in range(G):
        xg = x_ref[g].astype(jnp.bfloat16)
        e = lax.dot_general(xg, we, (((0,), (0,)), ((), ())),
                            preferred_element_type=jnp.float32)
        e = _silu(e + be)
        e3 = e.reshape(H, W, Cexp)

        zcol = jnp.zeros((H, pad, Cexp), jnp.float32)
        xp = jnp.concatenate([zcol, e3, zcol], axis=1)
        zrow = jnp.zeros((pad, W + 2 * pad, Cexp), jnp.float32)
        xp = jnp.concatenate([zrow, xp, zrow], axis=0)
        a = jnp.zeros((H, W, Cexp), jnp.float32)
        for j in range(K):
            col = xp[:, j:j + W, :]
            for i in range(K):
                tap = wd[i * K + j:i * K + j + 1, :]
                a = a + col[i:i + H] * tap
        a = _silu(a + bd)
        acts.append(a)
        pooled.append(jnp.sum(a, axis=(0, 1)).reshape(1, Cexp))

    p = jnp.concatenate(pooled, axis=0) * (1.0 / float(HW))
    s1 = _silu(jnp.dot(p, ws1_ref[...],
                       preferred_element_type=jnp.float32) + bs1_ref[...])
    s = jax.nn.sigmoid(jnp.dot(s1, ws2_ref[...],
                               preferred_element_type=jnp.float32)
                       + bs2_ref[...])

    wp = wp_ref[...]
    bp = bp_ref[...]
    for g in range(G):
        scaled = (acts[g] * s[g:g + 1].reshape(1, 1, Cexp))
        scaled = scaled.reshape(HW, Cexp).astype(jnp.bfloat16)
        o = lax.dot_general(wp, scaled, (((0,), (1,)), ((), ())),
                            preferred_element_type=jnp.float32)
        o_ref[g] = o + bp


def kernel(x, w_exp, b_exp, w_dw, b_dw, w_se1, b_se1, w_se2, b_se2,
           w_proj, b_proj):
    N, Cin, H, W = x.shape
    Cexp = w_exp.shape[1]
    Cout = w_proj.shape[1]
    Csq = w_se1.shape[1]
    K = w_dw.shape[0]
    HW = H * W
    G = 4 if N % 4 == 0 else 1

    xf = x.reshape(N, Cin, HW)
    out = pl.pallas_call(
        functools.partial(_fused_kernel, G=G, H=H, W=W, K=K),
        out_shape=jax.ShapeDtypeStruct((N, Cout, HW), x.dtype),
        grid_spec=pltpu.PrefetchScalarGridSpec(
            num_scalar_prefetch=0, grid=(N // G,),
            in_specs=[
                pl.BlockSpec((G, Cin, HW), lambda n: (n, 0, 0)),
                pl.BlockSpec((Cin, Cexp), lambda n: (0, 0)),
                pl.BlockSpec((1, Cexp), lambda n: (0, 0)),
                pl.BlockSpec((K * K, Cexp), lambda n: (0, 0)),
                pl.BlockSpec((1, Cexp), lambda n: (0, 0)),
                pl.BlockSpec((Cexp, Csq), lambda n: (0, 0)),
                pl.BlockSpec((1, Csq), lambda n: (0, 0)),
                pl.BlockSpec((Csq, Cexp), lambda n: (0, 0)),
                pl.BlockSpec((1, Cexp), lambda n: (0, 0)),
                pl.BlockSpec((Cexp, Cout), lambda n: (0, 0)),
                pl.BlockSpec((Cout, 1), lambda n: (0, 0)),
            ],
            out_specs=pl.BlockSpec((G, Cout, HW), lambda n: (n, 0, 0))),
        compiler_params=pltpu.CompilerParams(
            dimension_semantics=("parallel",),
            vmem_limit_bytes=64 * 1024 * 1024),
    )(xf, w_exp.astype(jnp.bfloat16), b_exp.reshape(1, Cexp),
      w_dw.reshape(K * K, Cexp), b_dw.reshape(1, Cexp),
      w_se1, b_se1.reshape(1, Csq), w_se2, b_se2.reshape(1, Cexp),
      w_proj.astype(jnp.bfloat16), b_proj.reshape(Cout, 1))
    return out.reshape(N, Cout, H, W)


# speedup vs baseline: 1.8943x; 1.4448x over previous
import functools

import jax
import jax.numpy as jnp
from jax import lax
from jax.experimental import pallas as pl
from jax.experimental.pallas import tpu as pltpu


def _silu(v):
    return v * jax.nn.sigmoid(v)


def _fused_kernel(x_ref, we_ref, be_ref, wd_ref, bd_ref,
                  ws1_ref, bs1_ref, ws2_ref, bs2_ref,
                  wp_ref, bp_ref, o_ref, *, G, H, W, K):
    HW = H * W
    Cexp = we_ref.shape[1]
    PADR = 48
    we = we_ref[...]
    be = be_ref[...]
    wd = wd_ref[...]
    bd = bd_ref[...]
    ones_row = jnp.ones((1, HW), jnp.bfloat16)

    PH = PADR + HW + PADR
    row = lax.broadcasted_iota(jnp.int32, (PH, Cexp), 0)
    xmod = (row + (W - PADR % W)) % W
    left_edge = xmod == 0
    right_edge = xmod == W - 1
    zrow = jnp.zeros((1, Cexp), jnp.bfloat16)
    zpad = jnp.zeros((PADR, Cexp), jnp.bfloat16)

    acts = []
    pooled = []
    for g in range(G):
        xg = x_ref[g].astype(jnp.bfloat16)
        e = lax.dot_general(xg, we, (((0,), (0,)), ((), ())),
                            preferred_element_type=jnp.float32)
        eb = _silu(e.astype(jnp.bfloat16) + be)

        ep = jnp.concatenate([zpad, eb, zpad], axis=0)
        cm = jnp.where(left_edge, jnp.bfloat16(0),
                       jnp.concatenate([zrow, ep[:-1]], axis=0))
        cp = jnp.where(right_edge, jnp.bfloat16(0),
                       jnp.concatenate([ep[1:], zrow], axis=0))
        cols = (cm, ep, cp)
        a = None
        for i in range(K):
            base = PADR + (i - (K - 1) // 2) * W
            for j in range(K):
                tap = wd[i * K + j:i * K + j + 1, :]
                t = cols[j][base:base + HW] * tap
                a = t if a is None else a + t
        a = _silu(a + bd)
        acts.append(a)
        pooled.append(jnp.dot(ones_row, a, preferred_element_type=jnp.float32))

    p = jnp.concatenate(pooled, axis=0) * (1.0 / float(HW))
    s1 = _silu(jnp.dot(p, ws1_ref[...],
                       preferred_element_type=jnp.float32) + bs1_ref[...])
    s = jax.nn.sigmoid(jnp.dot(s1, ws2_ref[...],
                               preferred_element_type=jnp.float32)
                       + bs2_ref[...])
    sb = s.astype(jnp.bfloat16)

    wp = wp_ref[...]
    bp = bp_ref[...]
    for g in range(G):
        scaled = acts[g] * sb[g:g + 1]
        o = lax.dot_general(wp, scaled, (((0,), (1,)), ((), ())),
                            preferred_element_type=jnp.float32)
        o_ref[g] = o + bp


def kernel(x, w_exp, b_exp, w_dw, b_dw, w_se1, b_se1, w_se2, b_se2,
           w_proj, b_proj):
    N, Cin, H, W = x.shape
    Cexp = w_exp.shape[1]
    Cout = w_proj.shape[1]
    Csq = w_se1.shape[1]
    K = w_dw.shape[0]
    HW = H * W
    G = 4 if N % 4 == 0 else 1
    bf16 = jnp.bfloat16

    xf = x.reshape(N, Cin, HW)
    out = pl.pallas_call(
        functools.partial(_fused_kernel, G=G, H=H, W=W, K=K),
        out_shape=jax.ShapeDtypeStruct((N, Cout, HW), x.dtype),
        grid_spec=pltpu.PrefetchScalarGridSpec(
            num_scalar_prefetch=0, grid=(N // G,),
            in_specs=[
                pl.BlockSpec((G, Cin, HW), lambda n: (n, 0, 0)),
                pl.BlockSpec((Cin, Cexp), lambda n: (0, 0)),
                pl.BlockSpec((1, Cexp), lambda n: (0, 0)),
                pl.BlockSpec((K * K, Cexp), lambda n: (0, 0)),
                pl.BlockSpec((1, Cexp), lambda n: (0, 0)),
                pl.BlockSpec((Cexp, Csq), lambda n: (0, 0)),
                pl.BlockSpec((1, Csq), lambda n: (0, 0)),
                pl.BlockSpec((Csq, Cexp), lambda n: (0, 0)),
                pl.BlockSpec((1, Cexp), lambda n: (0, 0)),
                pl.BlockSpec((Cexp, Cout), lambda n: (0, 0)),
                pl.BlockSpec((Cout, 1), lambda n: (0, 0)),
            ],
            out_specs=pl.BlockSpec((G, Cout, HW), lambda n: (n, 0, 0))),
        compiler_params=pltpu.CompilerParams(
            dimension_semantics=("parallel",),
            vmem_limit_bytes=64 * 1024 * 1024),
    )(xf, w_exp.astype(bf16), b_exp.reshape(1, Cexp).astype(bf16),
      w_dw.reshape(K * K, Cexp).astype(bf16),
      b_dw.reshape(1, Cexp).astype(bf16),
      w_se1, b_se1.reshape(1, Csq), w_se2, b_se2.reshape(1, Cexp),
      w_proj.astype(bf16), b_proj.reshape(Cout, 1))
    return out.reshape(N, Cout, H, W)
